# Initial kernel scaffold; baseline (speedup 1.0000x reference)
import jax
import jax.numpy as jnp
from jax.experimental import pallas as pl


def kernel(hyperedges, features_image, features_text, W1_image, b1_image, W2_image, b2_image, W1_text, b1_text, W2_text, b2_text, alpha):
    raise NotImplementedError("write your pallas kernel here")


# speedup vs baseline: 1.5715x; 1.5715x over previous
import functools

import jax
import jax.numpy as jnp
from jax import lax
from jax.experimental import pallas as pl
from jax.experimental.pallas import tpu as pltpu
from jax.experimental.pallas import tpu_sc as plsc

NC, NS, LANES = 2, 16, 16
NW = NC * NS


@functools.partial(jax.jit, static_argnums=(0, 1, 2, 3, 4))
def _pool_sc(E, K, N, D_IMG, D_TXT, idx_flat, feats_img, feats_txt):
    EW = E // NW
    CE = 16
    NCH = EW // CE
    ROWS = CE * K
    inv_k = 1.0 / K

    mesh = plsc.VectorSubcoreMesh(core_axis_name="c", subcore_axis_name="s",
                                  num_cores=NC, num_subcores=NS)

    def body(idx_hbm, img_hbm, txt_hbm, out_img, out_txt,
             idx_v, img_v, txt_v, pimg_v, ptxt_v, sem_i, sem_t):
        wid = lax.axis_index("s") * NC + lax.axis_index("c")
        ebase = wid * EW

        @pl.loop(0, NCH)
        def _chunk(c):
            e0 = ebase + c * CE
            pltpu.sync_copy(idx_hbm.at[pl.ds(e0 * K, ROWS)], idx_v)
            cp_i = pltpu.async_copy(img_hbm.at[idx_v], img_v, sem_i)
            cp_t = pltpu.async_copy(txt_hbm.at[idx_v], txt_v, sem_t)
            cp_i.wait()

            @pl.loop(0, CE)
            def _eimg(e):
                r0 = e * K
                for db in range(D_IMG // LANES):
                    sl = pl.ds(db * LANES, LANES)
                    acc = img_v[r0, sl]
                    for k in range(1, K):
                        acc = acc + img_v[r0 + k, sl]
                    pimg_v[e, sl] = acc * inv_k

            cp_t.wait()

            @pl.loop(0, CE)
            def _etxt(e):
                r0 = e * K
                for db in range(D_TXT // LANES):
                    sl = pl.ds(db * LANES, LANES)
                    acc = txt_v[r0, sl]
                    for k in range(1, K):
                        acc = acc + txt_v[r0 + k, sl]
                    ptxt_v[e, sl] = acc * inv_k

            pltpu.sync_copy(pimg_v, out_img.at[pl.ds(e0, CE)])
            pltpu.sync_copy(ptxt_v, out_txt.at[pl.ds(e0, CE)])

    fn = pl.kernel(
        body,
        out_type=(jax.ShapeDtypeStruct((E, D_IMG), jnp.float32),
                  jax.ShapeDtypeStruct((E, D_TXT), jnp.float32)),
        mesh=mesh,
        scratch_types=[
            pltpu.VMEM((ROWS,), jnp.int32),
            pltpu.VMEM((ROWS, D_IMG), jnp.float32),
            pltpu.VMEM((ROWS, D_TXT), jnp.float32),
            pltpu.VMEM((CE, D_IMG), jnp.float32),
            pltpu.VMEM((CE, D_TXT), jnp.float32),
            pltpu.SemaphoreType.DMA,
            pltpu.SemaphoreType.DMA,
        ],
    )
    return fn(idx_flat, feats_img, feats_txt)


def _mlp_body(scal_ref, pimg_ref, ptxt_ref, W1i_ref, b1i_ref, w2i_ref,
              W1t_ref, b1t_ref, w2t_ref, scores_ref, mask_ref):
    dn = (((1,), (0,)), ((), ()))
    hi = jnp.maximum(
        lax.dot_general(pimg_ref[...].astype(jnp.bfloat16), W1i_ref[...], dn,
                        preferred_element_type=jnp.float32) + b1i_ref[...], 0.0)
    si = jnp.sum(hi.astype(jnp.bfloat16).astype(jnp.float32)
                 * w2i_ref[...].astype(jnp.float32), axis=1) + scal_ref[2]
    ht = jnp.maximum(
        lax.dot_general(ptxt_ref[...].astype(jnp.bfloat16), W1t_ref[...], dn,
                        preferred_element_type=jnp.float32) + b1t_ref[...], 0.0)
    st = jnp.sum(ht.astype(jnp.bfloat16).astype(jnp.float32)
                 * w2t_ref[...].astype(jnp.float32), axis=1) + scal_ref[3]
    e = scal_ref[0] * si + scal_ref[1] * st
    sc = jax.nn.sigmoid(e)
    blk = sc.shape[0]
    scores_ref[...] = sc.reshape(1, 1, blk)
    mask_ref[...] = (sc > 0.5).astype(jnp.int32).reshape(1, 1, blk)


@functools.partial(jax.jit, static_argnums=(0, 1, 2, 3))
def _mlp_tc(E, D_IMG, D_TXT, H, scal, pooled_img, pooled_txt,
            W1i, b1i, w2i, W1t, b1t, w2t):
    BLK = 2048
    NB = E // BLK
    out = pl.pallas_call(
        _mlp_body,
        grid=(NB,),
        in_specs=[
            pl.BlockSpec(memory_space=pltpu.SMEM),
            pl.BlockSpec((BLK, D_IMG), lambda i: (i, 0)),
            pl.BlockSpec((BLK, D_TXT), lambda i: (i, 0)),
            pl.BlockSpec((D_IMG, H), lambda i: (0, 0)),
            pl.BlockSpec((1, H), lambda i: (0, 0)),
            pl.BlockSpec((1, H), lambda i: (0, 0)),
            pl.BlockSpec((D_TXT, H), lambda i: (0, 0)),
            pl.BlockSpec((1, H), lambda i: (0, 0)),
            pl.BlockSpec((1, H), lambda i: (0, 0)),
        ],
        out_specs=[
            pl.BlockSpec((1, 1, BLK), lambda i: (i, 0, 0)),
            pl.BlockSpec((1, 1, BLK), lambda i: (i, 0, 0)),
        ],
        out_shape=[
            jax.ShapeDtypeStruct((NB, 1, BLK), jnp.float32),
            jax.ShapeDtypeStruct((NB, 1, BLK), jnp.int32),
        ],
        compiler_params=pltpu.CompilerParams(
            dimension_semantics=("arbitrary",)),
    )(scal, pooled_img, pooled_txt, W1i, b1i, w2i, W1t, b1t, w2t)
    scores2d, mask2d = out
    return scores2d.reshape(E), mask2d.reshape(E)


def kernel(hyperedges, features_image, features_text, W1_image, b1_image,
           W2_image, b2_image, W1_text, b1_text, W2_text, b2_text, alpha):
    E, K = hyperedges.shape
    N, D_IMG = features_image.shape
    _, D_TXT = features_text.shape
    H = W1_image.shape[1]

    idx_flat = hyperedges.reshape(E * K).astype(jnp.int32)
    pooled_img, pooled_txt = _pool_sc(E, K, N, D_IMG, D_TXT,
                                      idx_flat, features_image, features_text)

    w = jax.nn.softmax(alpha, axis=0)
    scal = jnp.concatenate([w, b2_image, b2_text]).astype(jnp.float32)
    bf16 = jnp.bfloat16
    scores, mask_i = _mlp_tc(E, D_IMG, D_TXT, H, scal, pooled_img, pooled_txt,
                             W1_image.astype(bf16), b1_image.reshape(1, H),
                             W2_image.reshape(1, H).astype(bf16),
                             W1_text.astype(bf16), b1_text.reshape(1, H),
                             W2_text.reshape(1, H).astype(bf16))
    return (mask_i.astype(jnp.bool_), scores)


# speedup vs baseline: 1.8560x; 1.1810x over previous
import functools

import jax
import jax.numpy as jnp
from jax import lax
from jax.experimental import pallas as pl
from jax.experimental.pallas import tpu as pltpu
from jax.experimental.pallas import tpu_sc as plsc

NC, NS, LANES = 2, 16, 16
NW = NC * NS


@functools.partial(jax.jit, static_argnums=(0, 1, 2, 3, 4))
def _pool_sc(E, K, N, D_IMG, D_TXT, idx_flat, feats_img, feats_txt):
    EW = E // NW
    CE = 8
    NCH = EW // CE
    ROWS = CE * K
    NBUF = 2
    inv_k = 1.0 / K

    mesh = plsc.VectorSubcoreMesh(core_axis_name="c", subcore_axis_name="s",
                                  num_cores=NC, num_subcores=NS)

    def body(idx_hbm, img_hbm, txt_hbm, out_img, out_txt,
             idx_v, img_v, txt_v, pimg_v, ptxt_v,
             sem_i0, sem_i1, sem_t0, sem_t1):
        sems_i = (sem_i0, sem_i1)
        sems_t = (sem_t0, sem_t1)
        wid = lax.axis_index("s") * NC + lax.axis_index("c")
        ebase = wid * EW

        def start_fetch(cc, b):
            e0 = ebase + cc * CE
            pltpu.sync_copy(idx_hbm.at[pl.ds(e0 * K, ROWS)], idx_v.at[b])
            pltpu.async_copy(img_hbm.at[idx_v.at[b]], img_v.at[b], sems_i[b])
            pltpu.async_copy(txt_hbm.at[idx_v.at[b]], txt_v.at[b], sems_t[b])

        for b in range(NBUF):
            start_fetch(b, b)

        @pl.loop(0, NCH, step=NBUF)
        def _chunk(c):
            for b in range(NBUF):
                cc = c + b
                e0 = ebase + cc * CE
                pltpu.make_async_copy(img_hbm.at[idx_v.at[b]],
                                      img_v.at[b], sems_i[b]).wait()

                @pl.loop(0, CE)
                def _eimg(e):
                    r0 = e * K
                    for db in range(D_IMG // LANES):
                        sl = pl.ds(db * LANES, LANES)
                        acc = img_v[b, r0, sl]
                        for k in range(1, K):
                            acc = acc + img_v[b, r0 + k, sl]
                        pimg_v[e, sl] = acc * inv_k

                pltpu.make_async_copy(txt_hbm.at[idx_v.at[b]],
                                      txt_v.at[b], sems_t[b]).wait()

                @pl.loop(0, CE)
                def _etxt(e):
                    r0 = e * K
                    for db in range(D_TXT // LANES):
                        sl = pl.ds(db * LANES, LANES)
                        acc = txt_v[b, r0, sl]
                        for k in range(1, K):
                            acc = acc + txt_v[b, r0 + k, sl]
                        ptxt_v[e, sl] = acc * inv_k

                @pl.when(cc + NBUF < NCH)
                def _():
                    start_fetch(cc + NBUF, b)

                pltpu.sync_copy(pimg_v, out_img.at[pl.ds(e0, CE)])
                pltpu.sync_copy(ptxt_v, out_txt.at[pl.ds(e0, CE)])

    fn = pl.kernel(
        body,
        out_type=(jax.ShapeDtypeStruct((E, D_IMG), jnp.float32),
                  jax.ShapeDtypeStruct((E, D_TXT), jnp.float32)),
        mesh=mesh,
        scratch_types=[
            pltpu.VMEM((NBUF, ROWS), jnp.int32),
            pltpu.VMEM((NBUF, ROWS, D_IMG), jnp.float32),
            pltpu.VMEM((NBUF, ROWS, D_TXT), jnp.float32),
            pltpu.VMEM((CE, D_IMG), jnp.float32),
            pltpu.VMEM((CE, D_TXT), jnp.float32),
            pltpu.SemaphoreType.DMA,
            pltpu.SemaphoreType.DMA,
            pltpu.SemaphoreType.DMA,
            pltpu.SemaphoreType.DMA,
        ],
    )
    return fn(idx_flat, feats_img, feats_txt)


def _mlp_body(scal_ref, pimg_ref, ptxt_ref, W1i_ref, b1i_ref, w2i_ref,
              W1t_ref, b1t_ref, w2t_ref, scores_ref, mask_ref):
    dn = (((1,), (0,)), ((), ()))
    hi = jnp.maximum(
        lax.dot_general(pimg_ref[...].astype(jnp.bfloat16), W1i_ref[...], dn,
                        preferred_element_type=jnp.float32) + b1i_ref[...], 0.0)
    si = jnp.sum(hi.astype(jnp.bfloat16).astype(jnp.float32)
                 * w2i_ref[...].astype(jnp.float32), axis=1) + scal_ref[2]
    ht = jnp.maximum(
        lax.dot_general(ptxt_ref[...].astype(jnp.bfloat16), W1t_ref[...], dn,
                        preferred_element_type=jnp.float32) + b1t_ref[...], 0.0)
    st = jnp.sum(ht.astype(jnp.bfloat16).astype(jnp.float32)
                 * w2t_ref[...].astype(jnp.float32), axis=1) + scal_ref[3]
    e = scal_ref[0] * si + scal_ref[1] * st
    sc = jax.nn.sigmoid(e)
    blk = sc.shape[0]
    scores_ref[...] = sc.reshape(1, 1, blk)
    mask_ref[...] = (sc > 0.5).astype(jnp.int32).reshape(1, 1, blk)


@functools.partial(jax.jit, static_argnums=(0, 1, 2, 3))
def _mlp_tc(E, D_IMG, D_TXT, H, scal, pooled_img, pooled_txt,
            W1i, b1i, w2i, W1t, b1t, w2t):
    BLK = 2048
    NB = E // BLK
    out = pl.pallas_call(
        _mlp_body,
        grid=(NB,),
        in_specs=[
            pl.BlockSpec(memory_space=pltpu.SMEM),
            pl.BlockSpec((BLK, D_IMG), lambda i: (i, 0)),
            pl.BlockSpec((BLK, D_TXT), lambda i: (i, 0)),
            pl.BlockSpec((D_IMG, H), lambda i: (0, 0)),
            pl.BlockSpec((1, H), lambda i: (0, 0)),
            pl.BlockSpec((1, H), lambda i: (0, 0)),
            pl.BlockSpec((D_TXT, H), lambda i: (0, 0)),
            pl.BlockSpec((1, H), lambda i: (0, 0)),
            pl.BlockSpec((1, H), lambda i: (0, 0)),
        ],
        out_specs=[
            pl.BlockSpec((1, 1, BLK), lambda i: (i, 0, 0)),
            pl.BlockSpec((1, 1, BLK), lambda i: (i, 0, 0)),
        ],
        out_shape=[
            jax.ShapeDtypeStruct((NB, 1, BLK), jnp.float32),
            jax.ShapeDtypeStruct((NB, 1, BLK), jnp.int32),
        ],
        compiler_params=pltpu.CompilerParams(
            dimension_semantics=("arbitrary",)),
    )(scal, pooled_img, pooled_txt, W1i, b1i, w2i, W1t, b1t, w2t)
    scores2d, mask2d = out
    return scores2d.reshape(E), mask2d.reshape(E)


def kernel(hyperedges, features_image, features_text, W1_image, b1_image,
           W2_image, b2_image, W1_text, b1_text, W2_text, b2_text, alpha):
    E, K = hyperedges.shape
    N, D_IMG = features_image.shape
    _, D_TXT = features_text.shape
    H = W1_image.shape[1]

    idx_flat = hyperedges.reshape(E * K).astype(jnp.int32)
    pooled_img, pooled_txt = _pool_sc(E, K, N, D_IMG, D_TXT,
                                      idx_flat, features_image, features_text)

    w = jax.nn.softmax(alpha, axis=0)
    scal = jnp.concatenate([w, b2_image, b2_text]).astype(jnp.float32)
    bf16 = jnp.bfloat16
    scores, mask_i = _mlp_tc(E, D_IMG, D_TXT, H, scal, pooled_img, pooled_txt,
                             W1_image.astype(bf16), b1_image.reshape(1, H),
                             W2_image.reshape(1, H).astype(bf16),
                             W1_text.astype(bf16), b1_text.reshape(1, H),
                             W2_text.reshape(1, H).astype(bf16))
    return (mask_i.astype(jnp.bool_), scores)


# speedup vs baseline: 2.7661x; 1.4904x over previous
import functools

import jax
import jax.numpy as jnp
from jax import lax
from jax.experimental import pallas as pl
from jax.experimental.pallas import tpu as pltpu
from jax.experimental.pallas import tpu_sc as plsc

NC, NS, LANES = 2, 16, 16
NW = NC * NS


@functools.partial(jax.jit, static_argnums=(0, 1, 2, 3, 4))
def _pool_sc(E, K, N, D_IMG, D_TXT, idx_flat, feats_img, feats_txt):
    EW = E // NW
    CE = 8
    NCH = EW // CE
    ROWS = CE * K
    NBUF = 2
    inv_k = 1.0 / K

    mesh = plsc.VectorSubcoreMesh(core_axis_name="c", subcore_axis_name="s",
                                  num_cores=NC, num_subcores=NS)

    def _tree_sum(vals):
        while len(vals) > 1:
            nxt = [vals[i] + vals[i + 1] for i in range(0, len(vals) - 1, 2)]
            if len(vals) % 2:
                nxt.append(vals[-1])
            vals = nxt
        return vals[0]

    def body(idx_hbm, img_hbm, txt_hbm, out_img, out_txt,
             idx_all, img_v, txt_v, pimg_v, ptxt_v,
             sem_i0, sem_i1, sem_t0, sem_t1):
        sems_i = (sem_i0, sem_i1)
        sems_t = (sem_t0, sem_t1)
        wid = lax.axis_index("s") * NC + lax.axis_index("c")
        ebase = wid * EW
        pltpu.sync_copy(idx_hbm.at[pl.ds(ebase * K, EW * K)], idx_all)

        def start_fetch(cc, b):
            idx = idx_all.at[pl.ds(cc * ROWS, ROWS)]
            pltpu.async_copy(img_hbm.at[idx], img_v.at[b], sems_i[b])
            pltpu.async_copy(txt_hbm.at[idx], txt_v.at[b], sems_t[b])

        for b in range(NBUF):
            start_fetch(b, b)

        @pl.loop(0, NCH, step=NBUF)
        def _chunk(c):
            for b in range(NBUF):
                cc = c + b
                e0 = ebase + cc * CE
                idx_b = idx_all.at[pl.ds(cc * ROWS, ROWS)]
                pltpu.make_async_copy(img_hbm.at[idx_b],
                                      img_v.at[b], sems_i[b]).wait()

                @plsc.parallel_loop(0, CE, unroll=2)
                def _eimg(e):
                    r0 = e * K
                    for db in range(D_IMG // LANES):
                        sl = pl.ds(db * LANES, LANES)
                        acc = _tree_sum([img_v[b, r0 + k, sl]
                                         for k in range(K)])
                        pimg_v[e, sl] = acc * inv_k

                pltpu.make_async_copy(txt_hbm.at[idx_b],
                                      txt_v.at[b], sems_t[b]).wait()

                @plsc.parallel_loop(0, CE, unroll=2)
                def _etxt(e):
                    r0 = e * K
                    for db in range(D_TXT // LANES):
                        sl = pl.ds(db * LANES, LANES)
                        acc = _tree_sum([txt_v[b, r0 + k, sl]
                                         for k in range(K)])
                        ptxt_v[e, sl] = acc * inv_k

                @pl.when(cc + NBUF < NCH)
                def _():
                    start_fetch(cc + NBUF, b)

                pltpu.sync_copy(pimg_v, out_img.at[pl.ds(e0, CE)])
                pltpu.sync_copy(ptxt_v, out_txt.at[pl.ds(e0, CE)])

    fn = pl.kernel(
        body,
        out_type=(jax.ShapeDtypeStruct((E, D_IMG), jnp.float32),
                  jax.ShapeDtypeStruct((E, D_TXT), jnp.float32)),
        mesh=mesh,
        scratch_types=[
            pltpu.VMEM((EW * K,), jnp.int32),
            pltpu.VMEM((NBUF, ROWS, D_IMG), jnp.float32),
            pltpu.VMEM((NBUF, ROWS, D_TXT), jnp.float32),
            pltpu.VMEM((CE, D_IMG), jnp.float32),
            pltpu.VMEM((CE, D_TXT), jnp.float32),
            pltpu.SemaphoreType.DMA,
            pltpu.SemaphoreType.DMA,
            pltpu.SemaphoreType.DMA,
            pltpu.SemaphoreType.DMA,
        ],
    )
    return fn(idx_flat, feats_img, feats_txt)


def _mlp_body(scal_ref, pimg_ref, ptxt_ref, W1i_ref, b1i_ref, w2i_ref,
              W1t_ref, b1t_ref, w2t_ref, scores_ref, mask_ref):
    dn = (((1,), (0,)), ((), ()))
    hi = jnp.maximum(
        lax.dot_general(pimg_ref[...].astype(jnp.bfloat16), W1i_ref[...], dn,
                        preferred_element_type=jnp.float32) + b1i_ref[...], 0.0)
    si = jnp.sum(hi.astype(jnp.bfloat16).astype(jnp.float32)
                 * w2i_ref[...].astype(jnp.float32), axis=1) + scal_ref[2]
    ht = jnp.maximum(
        lax.dot_general(ptxt_ref[...].astype(jnp.bfloat16), W1t_ref[...], dn,
                        preferred_element_type=jnp.float32) + b1t_ref[...], 0.0)
    st = jnp.sum(ht.astype(jnp.bfloat16).astype(jnp.float32)
                 * w2t_ref[...].astype(jnp.float32), axis=1) + scal_ref[3]
    e = scal_ref[0] * si + scal_ref[1] * st
    sc = jax.nn.sigmoid(e)
    blk = sc.shape[0]
    scores_ref[...] = sc.reshape(1, 1, blk)
    mask_ref[...] = (sc > 0.5).astype(jnp.int32).reshape(1, 1, blk)


@functools.partial(jax.jit, static_argnums=(0, 1, 2, 3))
def _mlp_tc(E, D_IMG, D_TXT, H, scal, pooled_img, pooled_txt,
            W1i, b1i, w2i, W1t, b1t, w2t):
    BLK = 2048
    NB = E // BLK
    out = pl.pallas_call(
        _mlp_body,
        grid=(NB,),
        in_specs=[
            pl.BlockSpec(memory_space=pltpu.SMEM),
            pl.BlockSpec((BLK, D_IMG), lambda i: (i, 0)),
            pl.BlockSpec((BLK, D_TXT), lambda i: (i, 0)),
            pl.BlockSpec((D_IMG, H), lambda i: (0, 0)),
            pl.BlockSpec((1, H), lambda i: (0, 0)),
            pl.BlockSpec((1, H), lambda i: (0, 0)),
            pl.BlockSpec((D_TXT, H), lambda i: (0, 0)),
            pl.BlockSpec((1, H), lambda i: (0, 0)),
            pl.BlockSpec((1, H), lambda i: (0, 0)),
        ],
        out_specs=[
            pl.BlockSpec((1, 1, BLK), lambda i: (i, 0, 0)),
            pl.BlockSpec((1, 1, BLK), lambda i: (i, 0, 0)),
        ],
        out_shape=[
            jax.ShapeDtypeStruct((NB, 1, BLK), jnp.float32),
            jax.ShapeDtypeStruct((NB, 1, BLK), jnp.int32),
        ],
        compiler_params=pltpu.CompilerParams(
            dimension_semantics=("arbitrary",)),
    )(scal, pooled_img, pooled_txt, W1i, b1i, w2i, W1t, b1t, w2t)
    scores2d, mask2d = out
    return scores2d.reshape(E), mask2d.reshape(E)


def kernel(hyperedges, features_image, features_text, W1_image, b1_image,
           W2_image, b2_image, W1_text, b1_text, W2_text, b2_text, alpha):
    E, K = hyperedges.shape
    N, D_IMG = features_image.shape
    _, D_TXT = features_text.shape
    H = W1_image.shape[1]

    idx_flat = hyperedges.reshape(E * K).astype(jnp.int32)
    pooled_img, pooled_txt = _pool_sc(E, K, N, D_IMG, D_TXT,
                                      idx_flat, features_image, features_text)

    w = jax.nn.softmax(alpha, axis=0)
    scal = jnp.concatenate([w, b2_image, b2_text]).astype(jnp.float32)
    bf16 = jnp.bfloat16
    scores, mask_i = _mlp_tc(E, D_IMG, D_TXT, H, scal, pooled_img, pooled_txt,
                             W1_image.astype(bf16), b1_image.reshape(1, H),
                             W2_image.reshape(1, H).astype(bf16),
                             W1_text.astype(bf16), b1_text.reshape(1, H),
                             W2_text.reshape(1, H).astype(bf16))
    return (mask_i.astype(jnp.bool_), scores)
